# Initial kernel scaffold; baseline (speedup 1.0000x reference)
import jax
import jax.numpy as jnp
from jax.experimental import pallas as pl


def kernel(x, adj, inv_adj, W1, b1, W2, b2, W3, b3, W4, b4, W5, b5, W6, b6, W7, b7, W8, b8):
    raise NotImplementedError("write your pallas kernel here")


# speedup vs baseline: 1.3102x; 1.3102x over previous
import jax
import jax.numpy as jnp
from jax.experimental import pallas as pl

_N = 10000
_TM = 400
_NBLK = _N // _TM
_F32 = jnp.float32
_BF16 = jnp.bfloat16


def _lrelu(y):
    return jnp.where(y > 0, y, 0.01 * y)


def _sup1_body(x_ref, w_ref, o_ref):
    o_ref[...] = jnp.dot(
        x_ref[...], w_ref[...], preferred_element_type=_F32
    ).astype(_BF16)


def _layer1_body(a_ref, s_ref, w_ref, b_ref, a16_ref, sup_ref):
    a16 = a_ref[...].astype(_BF16)
    a16_ref[...] = a16
    y = jnp.dot(a16, s_ref[...], preferred_element_type=_F32) + b_ref[...]
    h = _lrelu(y)
    sup_ref[...] = jnp.dot(h, w_ref[...], preferred_element_type=_F32).astype(_BF16)


def _mid_body(a_ref, s_ref, w_ref, b_ref, sup_ref):
    y = jnp.dot(a_ref[...], s_ref[...], preferred_element_type=_F32) + b_ref[...]
    h = _lrelu(y)
    sup_ref[...] = jnp.dot(h, w_ref[...], preferred_element_type=_F32).astype(_BF16)


def _lat_body(a_ref, s_ref, w_ref, b_ref, lat_ref, sup_ref):
    y = jnp.dot(a_ref[...], s_ref[...], preferred_element_type=_F32) + b_ref[...]
    lat_ref[...] = y
    sup_ref[...] = jnp.dot(y, w_ref[...], preferred_element_type=_F32).astype(_BF16)


def _last_body(a_ref, s_ref, b_ref, out_ref):
    out_ref[...] = (
        jnp.dot(a_ref[...], s_ref[...], preferred_element_type=_F32) + b_ref[...]
    )


def _row_spec(d):
    return pl.BlockSpec((_TM, d), lambda i: (i, 0))


def _full_spec(r, c):
    return pl.BlockSpec((r, c), lambda i: (0, 0))


def kernel(x, adj, inv_adj, W1, b1, W2, b2, W3, b3, W4, b4, W5, b5, W6, b6,
           W7, b7, W8, b8):
    del inv_adj
    n, d0 = x.shape
    ws = [W1, W2, W3, W4, W5, W6, W7, W8]
    bs = [b.reshape(1, -1) for b in (b1, b2, b3, b4, b5, b6, b7, b8)]
    dims = [d0] + [w.shape[1] for w in ws]

    sup = pl.pallas_call(
        _sup1_body,
        grid=(_NBLK,),
        in_specs=[_row_spec(d0), _full_spec(d0, dims[1])],
        out_specs=_row_spec(dims[1]),
        out_shape=jax.ShapeDtypeStruct((n, dims[1]), _BF16),
    )(x, W1)

    adj16, sup = pl.pallas_call(
        _layer1_body,
        grid=(_NBLK,),
        in_specs=[
            _row_spec(n),
            _full_spec(n, dims[1]),
            _full_spec(dims[1], dims[2]),
            _full_spec(1, dims[1]),
        ],
        out_specs=[_row_spec(n), _row_spec(dims[2])],
        out_shape=[
            jax.ShapeDtypeStruct((n, n), _BF16),
            jax.ShapeDtypeStruct((n, dims[2]), _BF16),
        ],
    )(adj, sup, W2, bs[0])

    for li in (2, 3):
        sup = pl.pallas_call(
            _mid_body,
            grid=(_NBLK,),
            in_specs=[
                _row_spec(n),
                _full_spec(n, dims[li]),
                _full_spec(dims[li], dims[li + 1]),
                _full_spec(1, dims[li]),
            ],
            out_specs=_row_spec(dims[li + 1]),
            out_shape=jax.ShapeDtypeStruct((n, dims[li + 1]), _BF16),
        )(adj16, sup, ws[li], bs[li - 1])

    lat, sup = pl.pallas_call(
        _lat_body,
        grid=(_NBLK,),
        in_specs=[
            _row_spec(n),
            _full_spec(n, dims[4]),
            _full_spec(dims[4], dims[5]),
            _full_spec(1, dims[4]),
        ],
        out_specs=[_row_spec(dims[4]), _row_spec(dims[5])],
        out_shape=[
            jax.ShapeDtypeStruct((n, dims[4]), _F32),
            jax.ShapeDtypeStruct((n, dims[5]), _BF16),
        ],
    )(adj16, sup, W5, bs[3])

    for li in (5, 6, 7):
        sup = pl.pallas_call(
            _mid_body,
            grid=(_NBLK,),
            in_specs=[
                _row_spec(n),
                _full_spec(n, dims[li]),
                _full_spec(dims[li], dims[li + 1]),
                _full_spec(1, dims[li]),
            ],
            out_specs=_row_spec(dims[li + 1]),
            out_shape=jax.ShapeDtypeStruct((n, dims[li + 1]), _BF16),
        )(adj16, sup, ws[li], bs[li - 1])

    out = pl.pallas_call(
        _last_body,
        grid=(_NBLK,),
        in_specs=[_row_spec(n), _full_spec(n, dims[8]), _full_spec(1, dims[8])],
        out_specs=_row_spec(dims[8]),
        out_shape=jax.ShapeDtypeStruct((n, dims[8]), _F32),
    )(adj16, sup, bs[7])

    return (lat, out)
